# baseline (speedup 1.0000x reference)
import functools

import jax
import jax.numpy as jnp
from jax import lax
from jax.experimental import pallas as pl
from jax.experimental.pallas import tpu as pltpu
from jax.experimental.pallas import tpu_sc as plsc

D = 64
NC, NS = 2, 16
NW = NC * NS


def _sc_gather(u_emb, v_emb, pos_u, vidx, bpw, nv):
    B = pos_u.shape[0]
    mesh = plsc.VectorSubcoreMesh(
        core_axis_name="c", subcore_axis_name="s", num_cores=NC, num_subcores=NS
    )

    @functools.partial(
        pl.kernel,
        out_type=[
            jax.ShapeDtypeStruct((B, D), jnp.float32),
            jax.ShapeDtypeStruct((NW, nv, bpw, D), jnp.float32),
        ],
        mesh=mesh,
        compiler_params=pltpu.CompilerParams(use_tc_tiling_on_sc=False),
        scratch_types=[
            pltpu.VMEM((bpw,), jnp.int32),
            pltpu.VMEM((nv, bpw), jnp.int32),
            pltpu.VMEM((bpw, D), jnp.float32),
            pltpu.VMEM((nv, bpw, D), jnp.float32),
            pltpu.SemaphoreType.DMA,
        ],
    )
    def sc_kernel(u_hbm, v_hbm, posu_hbm, vidx_hbm, outu_hbm, outv_hbm,
                  idxu, idxv, rowsu, rowsv, sem):
        wid = lax.axis_index("s") * NC + lax.axis_index("c")
        base = wid * bpw
        pltpu.sync_copy(posu_hbm.at[pl.ds(base, bpw)], idxu)
        pltpu.sync_copy(vidx_hbm.at[wid], idxv)
        copies = [pltpu.async_copy(u_hbm.at[idxu], rowsu, sem)]
        for j in range(nv):
            copies.append(pltpu.async_copy(v_hbm.at[idxv.at[j]], rowsv.at[j], sem))
        for c in copies:
            c.wait()
        pltpu.sync_copy(rowsu, outu_hbm.at[pl.ds(base, bpw)])
        pltpu.sync_copy(rowsv, outv_hbm.at[wid])

    return sc_kernel(u_emb, v_emb, pos_u, vidx)


def _tc_score(rows_u, rows_v, dur_w, dur_b, bpw, nv, dur_from_v):
    B = rows_u.shape[0]

    def body(u_ref, v_ref, w_ref, b_ref, loss_ref, dur_ref):
        w = pl.program_id(0)
        u = u_ref[...]
        pv = v_ref[0:bpw, :]
        s = jnp.clip(jnp.sum(u * pv, axis=1, keepdims=True), -10.0, 10.0)
        tot = jnp.log1p(jnp.exp(-s))
        for j in range(1, nv):
            nvr = v_ref[j * bpw:(j + 1) * bpw, :]
            ns = jnp.clip(jnp.sum(u * nvr, axis=1, keepdims=True), -10.0, 10.0)
            tot = tot + jnp.log1p(jnp.exp(ns))
        part = jnp.sum(tot)

        @pl.when(w == 0)
        def _init():
            loss_ref[0] = 0.0

        loss_ref[0] += part

        @pl.when(w == NW - 1)
        def _finish():
            loss_ref[0] = loss_ref[0] / B

        sel = pv if dur_from_v else u
        dur_ref[...] = jnp.sum(sel * w_ref[...], axis=1, keepdims=True) + b_ref[0]

    return pl.pallas_call(
        body,
        grid=(NW,),
        in_specs=[
            pl.BlockSpec((bpw, D), lambda w: (w, 0)),
            pl.BlockSpec((nv * bpw, D), lambda w: (w, 0)),
            pl.BlockSpec((1, D), lambda w: (0, 0)),
            pl.BlockSpec(memory_space=pltpu.SMEM),
        ],
        out_specs=[
            pl.BlockSpec(memory_space=pltpu.SMEM),
            pl.BlockSpec((bpw, 1), lambda w: (w, 0)),
        ],
        out_shape=[
            jax.ShapeDtypeStruct((1,), jnp.float32),
            jax.ShapeDtypeStruct((B, 1), jnp.float32),
        ],
    )(rows_u, rows_v, dur_w, dur_b)


def kernel(pos_u, pos_v, neg_v, predict_fix, u_emb, v_emb, dur_w, dur_b):
    B = pos_u.shape[0]
    nneg = neg_v.shape[1]
    nv = 1 + nneg
    bpw = B // NW

    negt = jnp.transpose(neg_v.reshape(NW, bpw, nneg), (0, 2, 1))
    vidx = jnp.concatenate([pos_v.reshape(NW, 1, bpw), negt], axis=1)

    rows_u, rows_v = _sc_gather(u_emb, v_emb, pos_u, vidx, bpw, nv)

    dur_from_v = isinstance(predict_fix, str) and predict_fix == "output"
    loss, dur = _tc_score(
        rows_u, rows_v.reshape(NW * nv * bpw, D), dur_w, dur_b, bpw, nv, dur_from_v
    )
    return loss[0], dur.reshape(B)


# speedup vs baseline: 1.5700x; 1.5700x over previous
import functools

import jax
import jax.numpy as jnp
from jax import lax
from jax.experimental import pallas as pl
from jax.experimental.pallas import tpu as pltpu
from jax.experimental.pallas import tpu_sc as plsc

D = 64
NC, NS = 2, 16
NW = NC * NS


def _sc_gather(u_emb, v_emb, pos_u, vidx, bpw, nv):
    B = pos_u.shape[0]
    mesh = plsc.VectorSubcoreMesh(
        core_axis_name="c", subcore_axis_name="s", num_cores=NC, num_subcores=NS
    )

    @functools.partial(
        pl.kernel,
        out_type=[
            jax.ShapeDtypeStruct((B, D), jnp.float32),
            jax.ShapeDtypeStruct((NW, nv, bpw, D), jnp.float32),
        ],
        mesh=mesh,
        compiler_params=pltpu.CompilerParams(use_tc_tiling_on_sc=True),
        scratch_types=[
            pltpu.VMEM((bpw,), jnp.int32),
            pltpu.VMEM((nv, bpw), jnp.int32),
            pltpu.VMEM((bpw, D), jnp.float32),
            pltpu.VMEM((nv, bpw, D), jnp.float32),
            pltpu.SemaphoreType.DMA,
        ],
    )
    def sc_kernel(u_hbm, v_hbm, posu_hbm, vidx_hbm, outu_hbm, outv_hbm,
                  idxu, idxv, rowsu, rowsv, sem):
        wid = lax.axis_index("s") * NC + lax.axis_index("c")
        base = wid * bpw
        pltpu.sync_copy(posu_hbm.at[pl.ds(base, bpw)], idxu)
        pltpu.sync_copy(vidx_hbm.at[wid], idxv)

        def u_group(g, carry):
            vec = idxu[pl.ds(g * 16, 16)]
            for k in range(16):
                i = g * 16 + k
                pltpu.async_copy(u_hbm.at[pl.ds(vec[k], 1), :],
                                 rowsu.at[pl.ds(i, 1), :], sem)
            return carry

        lax.fori_loop(0, bpw // 16, u_group, 0)

        for j in range(nv):
            def v_group(g, carry, j=j):
                vec = idxv[j, pl.ds(g * 16, 16)]
                for k in range(16):
                    i = g * 16 + k
                    pltpu.async_copy(v_hbm.at[pl.ds(vec[k], 1), :],
                                     rowsv.at[j].at[pl.ds(i, 1), :], sem)
                return carry

            lax.fori_loop(0, bpw // 16, v_group, 0)

        pltpu.make_async_copy(u_hbm.at[pl.ds(0, bpw), :], rowsu, sem).wait()
        for j in range(nv):
            pltpu.make_async_copy(v_hbm.at[pl.ds(0, bpw), :], rowsv.at[j], sem).wait()

        pltpu.sync_copy(rowsu, outu_hbm.at[pl.ds(base, bpw)])
        pltpu.sync_copy(rowsv, outv_hbm.at[wid])

    return sc_kernel(u_emb, v_emb, pos_u, vidx)


def _tc_score(rows_u, rows_v, dur_w, dur_b, bpw, nv, dur_from_v):
    B = rows_u.shape[0]

    def body(u_ref, v_ref, w_ref, b_ref, loss_ref, dur_ref):
        w = pl.program_id(0)
        u = u_ref[...]
        pv = v_ref[0:bpw, :]
        s = jnp.clip(jnp.sum(u * pv, axis=1, keepdims=True), -10.0, 10.0)
        tot = jnp.log1p(jnp.exp(-s))
        for j in range(1, nv):
            nvr = v_ref[j * bpw:(j + 1) * bpw, :]
            ns = jnp.clip(jnp.sum(u * nvr, axis=1, keepdims=True), -10.0, 10.0)
            tot = tot + jnp.log1p(jnp.exp(ns))
        part = jnp.sum(tot)

        @pl.when(w == 0)
        def _init():
            loss_ref[0] = 0.0

        loss_ref[0] += part

        @pl.when(w == NW - 1)
        def _finish():
            loss_ref[0] = loss_ref[0] / B

        sel = pv if dur_from_v else u
        dur_ref[...] = jnp.sum(sel * w_ref[...], axis=1, keepdims=True) + b_ref[0]

    return pl.pallas_call(
        body,
        grid=(NW,),
        in_specs=[
            pl.BlockSpec((bpw, D), lambda w: (w, 0)),
            pl.BlockSpec((nv * bpw, D), lambda w: (w, 0)),
            pl.BlockSpec((1, D), lambda w: (0, 0)),
            pl.BlockSpec(memory_space=pltpu.SMEM),
        ],
        out_specs=[
            pl.BlockSpec(memory_space=pltpu.SMEM),
            pl.BlockSpec((bpw, 1), lambda w: (w, 0)),
        ],
        out_shape=[
            jax.ShapeDtypeStruct((1,), jnp.float32),
            jax.ShapeDtypeStruct((B, 1), jnp.float32),
        ],
    )(rows_u, rows_v, dur_w, dur_b)


def kernel(pos_u, pos_v, neg_v, predict_fix, u_emb, v_emb, dur_w, dur_b):
    B = pos_u.shape[0]
    nneg = neg_v.shape[1]
    nv = 1 + nneg
    bpw = B // NW

    negt = jnp.transpose(neg_v.reshape(NW, bpw, nneg), (0, 2, 1))
    vidx = jnp.concatenate([pos_v.reshape(NW, 1, bpw), negt], axis=1)

    rows_u, rows_v = _sc_gather(u_emb, v_emb, pos_u, vidx, bpw, nv)

    dur_from_v = isinstance(predict_fix, str) and predict_fix == "output"
    loss, dur = _tc_score(
        rows_u, rows_v.reshape(NW * nv * bpw, D), dur_w, dur_b, bpw, nv, dur_from_v
    )
    return loss[0], dur.reshape(B)
